# baseline (speedup 1.0000x reference)
import functools

import jax
import jax.numpy as jnp
from jax import lax
from jax.experimental import pallas as pl
from jax.experimental.pallas import tpu as pltpu
from jax.experimental.pallas import tpu_sc as plsc

_N = 2048
_DEG = 16
_EPS = 1e-5
_RB = 256
_KS = _N // _RB
_NC, _NS = 2, 16
_NW = _NC * _NS
_NPW = _N // _NW


def _pad128(d):
    return ((d + 127) // 128) * 128


def _pad256(d):
    return ((d + 255) // 256) * 256


def _seq_vreg_sum(x, init=None):
    acc = init
    for c in range(0, x.shape[0], 8):
        blk = x[c:c + 8]
        acc = blk if acc is None else acc + blk
    return acc


def _butterfly(acc):
    y = acc + jnp.concatenate([acc[4:], acc[:4]], axis=0)
    z = y + jnp.concatenate([y[2:], y[:2]], axis=0)
    w = z + jnp.concatenate([z[1:], z[:1]], axis=0)
    return w[0:1]


@functools.lru_cache(maxsize=None)
def _make_conv(P, H1, H2):
    f32 = jnp.float32

    def body(A_ref, X_ref,
             W1a_ref, b1a_ref, W2a_ref, b2a_ref, ga_ref, bta_ref,
             W1b_ref, b1b_ref, W2b_ref, b2b_ref, gb_ref, btb_ref,
             y1_ref, y2_ref,
             y1s, y2s, s1, s2):
        j = pl.program_id(0)

        agg = jnp.dot(A_ref[...], X_ref[...], preferred_element_type=f32)
        t1 = jnp.maximum(
            jnp.dot(agg, W1a_ref[...], preferred_element_type=f32)
            + b1a_ref[...], 0.0)
        r1 = jnp.dot(t1, W2a_ref[...], preferred_element_type=f32) + b2a_ref[...]
        t2 = jnp.maximum(
            jnp.dot(agg, W1b_ref[...], preferred_element_type=f32)
            + b1b_ref[...], 0.0)
        r2 = jnp.dot(t2, W2b_ref[...], preferred_element_type=f32) + b2b_ref[...]
        y1s[pl.ds(j * _RB, _RB), :] = r1
        y2s[pl.ds(j * _RB, _RB), :] = r2

        @pl.when(j == 0)
        def _():
            s1[...] = jnp.zeros_like(s1)
            s2[...] = jnp.zeros_like(s2)

        s1[...] = _seq_vreg_sum(r1, s1[...])
        s2[...] = _seq_vreg_sum(r2, s2[...])

        @pl.when(j == _KS - 1)
        def _():
            inv_n = 1.0 / _N
            m1 = _butterfly(s1[...]) * inv_n
            c1 = y1s[...] - m1
            v1 = _butterfly(_seq_vreg_sum(c1 * c1)) * inv_n
            y1_ref[...] = jnp.maximum(
                c1 / jnp.sqrt(v1 + _EPS) * ga_ref[...] + bta_ref[...], 0.0)
            m2 = _butterfly(s2[...]) * inv_n
            c2 = y2s[...] - m2
            v2 = _butterfly(_seq_vreg_sum(c2 * c2)) * inv_n
            y2_ref[...] = jnp.maximum(
                c2 / jnp.sqrt(v2 + _EPS) * gb_ref[...] + btb_ref[...], 0.0)

    def whole(shape):
        nd = len(shape)
        return pl.BlockSpec(shape, lambda j, _nd=nd: (0,) * _nd)

    return pl.pallas_call(
        body,
        grid=(_KS,),
        in_specs=[
            pl.BlockSpec((_RB, _N), lambda j: (j, 0)),
            whole((_N, P)),
            whole((P, H1)), whole((1, H1)), whole((H1, 128)), whole((1, 128)),
            whole((1, 128)), whole((1, 128)),
            whole((P, H2)), whole((1, H2)), whole((H2, 64)), whole((1, 64)),
            whole((1, 64)), whole((1, 64)),
        ],
        out_specs=[whole((_N, 128)), whole((_N, 64))],
        out_shape=[
            jax.ShapeDtypeStruct((_N, 128), f32),
            jax.ShapeDtypeStruct((_N, 64), f32),
        ],
        scratch_shapes=[
            pltpu.VMEM((_N, 128), f32), pltpu.VMEM((_N, 64), f32),
            pltpu.VMEM((8, 128), f32), pltpu.VMEM((8, 64), f32),
        ],
        compiler_params=pltpu.CompilerParams(
            dimension_semantics=("arbitrary",)),
    )


@functools.lru_cache(maxsize=None)
def _make_tablize(d, P):
    f32 = jnp.float32

    def body(X_ref, table_ref):
        X = X_ref[...]
        cmin = jnp.min(X, axis=0, keepdims=True)
        table_ref[...] = jnp.concatenate([X, cmin], axis=0)

    return pl.pallas_call(
        body,
        out_shape=jax.ShapeDtypeStruct((_N + 1, P), f32),
    )


@functools.lru_cache(maxsize=None)
def _make_maxpool(P):
    f32 = jnp.float32
    npc = 8 if (_DEG * P * 4 * 2 * 8) <= 420000 else 4
    gs = npc * _DEG
    nch = _NPW // npc
    cpw = 8 // npc
    mesh = plsc.VectorSubcoreMesh(core_axis_name="c", subcore_axis_name="s")

    @functools.partial(
        pl.kernel,
        mesh=mesh,
        out_type=jax.ShapeDtypeStruct((_N, P), f32),
        scratch_types=[
            pltpu.VMEM((gs, P), f32),
            pltpu.VMEM((gs, P), f32),
            pltpu.VMEM((gs,), jnp.int32),
            pltpu.VMEM((gs,), jnp.int32),
            pltpu.VMEM((8, P), f32),
            pltpu.SemaphoreType.DMA,
            pltpu.SemaphoreType.DMA,
        ],
    )
    def mp(table_hbm, pnl_hbm, out_hbm, rows0, rows1, idx0, idx1, obuf,
           sem0, sem1):
        wid = lax.axis_index("s") * _NC + lax.axis_index("c")
        node0 = wid * _NPW
        rows = (rows0, rows1)
        idxs = (idx0, idx1)
        sems = (sem0, sem1)

        def start(g, b):
            pltpu.sync_copy(
                pnl_hbm.at[pl.ds((node0 + g * npc) * _DEG, gs)], idxs[b])
            pltpu.async_copy(table_hbm.at[idxs[b]], rows[b], sems[b])

        start(0, 0)
        for g in range(nch):
            b = g % 2
            pltpu.make_async_copy(table_hbm.at[idxs[b]], rows[b],
                                  sems[b]).wait()
            if g + 1 < nch:
                start(g + 1, 1 - b)
            r = rows[b]
            orow = (g % cpw) * npc
            for p in range(npc):
                def col(ci, carry, _p=p, _orow=orow):
                    c0 = ci * 16
                    a = r[_p * _DEG, pl.ds(c0, 16)]
                    for k in range(1, _DEG):
                        a = jnp.maximum(a, r[_p * _DEG + k, pl.ds(c0, 16)])
                    obuf[_orow + _p, pl.ds(c0, 16)] = a
                    return carry
                lax.fori_loop(0, P // 16, col, 0)
            if (g + 1) % cpw == 0:
                pltpu.sync_copy(
                    obuf, out_hbm.at[pl.ds(node0 + (g + 1 - cpw) * npc, 8)])

    return mp


@functools.lru_cache(maxsize=None)
def _make_head(P5):
    f32 = jnp.float32

    def body(X_ref, W1_ref, b1_ref, al_ref, W2_ref, b2_ref, out_ref):
        pooled = _butterfly(_seq_vreg_sum(X_ref[...])) * (1.0 / _N)
        z = jnp.dot(pooled, W1_ref[...], preferred_element_type=f32) + b1_ref[...]
        z = jnp.where(z > 0.0, z, al_ref[...] * z)
        r = jnp.dot(z, W2_ref[...], preferred_element_type=f32) + b2_ref[...]
        rp = jnp.concatenate([r, jnp.zeros((1, 126), f32)], axis=1)
        out_ref[...] = jnp.concatenate([rp, jnp.zeros((7, 128), f32)], axis=0)

    return pl.pallas_call(
        body,
        out_shape=jax.ShapeDtypeStruct((8, 128), f32),
    )


def kernel(h, A, padded_neighbor_list,
           gc1_W1_0, gc1_b1_0, gc1_W2_0, gc1_b2_0, gc1_gamma_0, gc1_beta_0,
           gc2_W1_0, gc2_b1_0, gc2_W2_0, gc2_b2_0, gc2_gamma_0, gc2_beta_0,
           gc1_W1_1, gc1_b1_1, gc1_W2_1, gc1_b2_1, gc1_gamma_1, gc1_beta_1,
           gc2_W1_1, gc2_b1_1, gc2_W2_1, gc2_b2_1, gc2_gamma_1, gc2_beta_1,
           gc1_W1_2, gc1_b1_2, gc1_W2_2, gc1_b2_2, gc1_gamma_2, gc1_beta_2,
           gc2_W1_2, gc2_b1_2, gc2_W2_2, gc2_b2_2, gc2_gamma_2, gc2_beta_2,
           gc1_W1_3, gc1_b1_3, gc1_W2_3, gc1_b2_3, gc1_gamma_3, gc1_beta_3,
           gc2_W1_3, gc2_b1_3, gc2_W2_3, gc2_b2_3, gc2_gamma_3, gc2_beta_3,
           gc1_W1_4, gc1_b1_4, gc1_W2_4, gc1_b2_4, gc1_gamma_4, gc1_beta_4,
           gc2_W1_4, gc2_b1_4, gc2_W2_4, gc2_b2_4, gc2_gamma_4, gc2_beta_4,
           Wc1, bc1, alpha, Wc2, bc2):
    prm = dict(locals())
    f32 = jnp.float32
    pnl_flat = padded_neighbor_list.astype(jnp.int32).reshape(-1)

    d = 80
    P = _pad128(d)
    X = jnp.concatenate(
        [h[0].astype(f32), jnp.zeros((_N, P - d), f32)], axis=1)

    for i in range(5):
        W1a = prm['gc1_W1_%d' % i]
        h1 = W1a.shape[1]
        H1 = _pad128(h1)
        W1b = prm['gc2_W1_%d' % i]
        h2 = W1b.shape[1]
        H2 = _pad128(h2)
        conv = _make_conv(P, H1, H2)
        y1, y2 = conv(
            A, X,
            jnp.pad(W1a, ((0, P - d), (0, H1 - h1))),
            jnp.pad(prm['gc1_b1_%d' % i], (0, H1 - h1)).reshape(1, -1),
            jnp.pad(prm['gc1_W2_%d' % i], ((0, H1 - h1), (0, 0))),
            prm['gc1_b2_%d' % i].reshape(1, -1),
            prm['gc1_gamma_%d' % i].reshape(1, -1),
            prm['gc1_beta_%d' % i].reshape(1, -1),
            jnp.pad(W1b, ((0, P - d), (0, H2 - h2))),
            jnp.pad(prm['gc2_b1_%d' % i], (0, H2 - h2)).reshape(1, -1),
            jnp.pad(prm['gc2_W2_%d' % i], ((0, H2 - h2), (0, 0))),
            prm['gc2_b2_%d' % i].reshape(1, -1),
            prm['gc2_gamma_%d' % i].reshape(1, -1),
            prm['gc2_beta_%d' % i].reshape(1, -1),
        )
        table = _make_tablize(d, P)(X)
        o1 = _make_maxpool(P)(table, pnl_flat)

        dn = d + 192
        Pn = _pad128(dn)
        X = jnp.concatenate(
            [o1[:, :d], y1, y2,
             jnp.zeros((_N, Pn - dn), f32)], axis=1)
        d, P = dn, Pn

    head = _make_head(P)
    res = head(X, jnp.pad(Wc1, ((0, P - d), (0, 0))),
               bc1.reshape(1, -1), alpha.reshape(1, -1),
               Wc2, bc2.reshape(1, -1))
    return res[0:1, 0:2]
